# baseline (speedup 1.0000x reference)
import jax
import jax.numpy as jnp
from jax.experimental import pallas as pl
from jax.experimental.pallas import tpu as pltpu

_N = 2048
_D = 64
_H = 8
_NB = 8
_DH = _D * _H
_C = _H * _NB
_LOG2E = 1.4426950408889634
_SLOPE = 0.2
_NEG = -1e9
_BI = 256


def _gat_kernel(emb_ref, edges_ref, wq_ref, bqr_ref, wk_ref, bkr_ref,
                wv_ref, bv_ref, asrc_ref, adst_ref, wp_ref, bp_ref,
                out_ref, val_s, ssrc_s, e1_s, e2_s):
    step = pl.program_id(0)

    @pl.when(step == 0)
    def _init():
        emb = emb_ref[...]
        val_s[...] = (
            jnp.dot(emb, wv_ref[...], preferred_element_type=jnp.float32)
            + bv_ref[...])
        asrc = asrc_ref[...]
        adst = adst_ref[...]
        wq = wq_ref[...]
        wk = wk_ref[...]
        cq_blocks = []
        ck_blocks = []
        for h in range(_H):
            ah = asrc[h * _NB:(h + 1) * _NB, :]
            dh = adst[h * _NB:(h + 1) * _NB, :]
            wq_h = wq[:, h * _D:(h + 1) * _D]
            wk_h = wk[:, h * _D:(h + 1) * _D]
            cq_blocks.append(jax.lax.dot_general(
                wq_h, ah, (((1,), (1,)), ((), ())),
                preferred_element_type=jnp.float32))
            ck_blocks.append(jax.lax.dot_general(
                wk_h, dh, (((1,), (1,)), ((), ())),
                preferred_element_type=jnp.float32))
        cq = jnp.concatenate(cq_blocks, axis=1)
        ck = jnp.concatenate(ck_blocks, axis=1)
        cq_b = jnp.sum(bqr_ref[...] * asrc, axis=1, keepdims=True)
        ck_b = jnp.sum(bkr_ref[...] * adst, axis=1, keepdims=True)
        ssrc_s[...] = jnp.dot(emb, cq,
                              preferred_element_type=jnp.float32) * _LOG2E
        sdstt = (jax.lax.dot_general(ck, emb, (((0,), (1,)), ((), ())),
                                     preferred_element_type=jnp.float32)
                 + cq_b + ck_b) * _LOG2E
        mshift = jnp.concatenate(
            [jnp.broadcast_to(
                jnp.max(sdstt[h * _NB:(h + 1) * _NB, :], keepdims=True),
                (_NB, _N))
             for h in range(_H)], axis=0)
        e1_s[...] = jnp.exp2(sdstt - mshift)
        e2_s[...] = jnp.exp2(_SLOPE * sdstt - mshift)

    e = edges_ref[...]
    ssrc = ssrc_s[pl.ds(step * _BI, _BI), :]
    isrc = e & 15
    idst = e & 7
    neg_half = jnp.full((_BI, _NB), -7e8, jnp.float32)
    acc = jnp.zeros((_BI, _D), jnp.float32)
    for h in range(_H):
        ta = ssrc[:, h * _NB:(h + 1) * _NB]
        rmax = jnp.max(ta, axis=1, keepdims=True)
        ta16 = jnp.concatenate([ta, neg_half], axis=1)
        pa1 = jnp.exp2(ta16 - rmax)
        pa2 = jnp.exp2(_SLOPE * ta16 - rmax)
        ga1 = jnp.take_along_axis(pa1, isrc, axis=1)
        ga2 = jnp.take_along_axis(pa2, isrc, axis=1)
        gb1 = jnp.take_along_axis(e1_s[h * _NB:(h + 1) * _NB, :],
                                  idst, axis=0)
        gb2 = jnp.take_along_axis(e2_s[h * _NB:(h + 1) * _NB, :],
                                  idst, axis=0)
        p = jnp.maximum(ga1 * gb1, ga2 * gb2)
        inv = 1.0 / jnp.maximum(jnp.sum(p, axis=1, keepdims=True), 1e-30)
        oh = jnp.dot(p, val_s[:, h * _D:(h + 1) * _D],
                     preferred_element_type=jnp.float32) * inv
        acc = acc + jnp.dot(oh, wp_ref[h * _D:(h + 1) * _D, :],
                            preferred_element_type=jnp.float32)
    out_ref[...] = acc + bp_ref[...]


def kernel(atom_embeddings, edges, Wq, bq, Wk, bk, Wv, bv, a_src, a_dst,
           W_proj, b_proj):
    asrc2 = a_src.transpose(1, 0, 2).reshape(_C, _D)
    adst2 = a_dst.transpose(1, 0, 2).reshape(_C, _D)
    bq_rep = jnp.broadcast_to(
        bq.reshape(_H, 1, _D), (_H, _NB, _D)).reshape(_C, _D)
    bk_rep = jnp.broadcast_to(
        bk.reshape(_H, 1, _D), (_H, _NB, _D)).reshape(_C, _D)
    bv2 = bv.reshape(1, _DH)
    bp2 = b_proj.reshape(1, _D)

    full = lambda shape: pl.BlockSpec(shape, lambda i: (0,) * len(shape))
    out = pl.pallas_call(
        _gat_kernel,
        grid=(_N // _BI,),
        in_specs=[
            full((_N, _D)),
            pl.BlockSpec((_BI, _N), lambda i: (i, 0)),
            full((_D, _DH)),
            full((_C, _D)),
            full((_D, _DH)),
            full((_C, _D)),
            full((_D, _DH)),
            full((1, _DH)),
            full((_C, _D)),
            full((_C, _D)),
            full((_DH, _D)),
            full((1, _D)),
        ],
        out_specs=pl.BlockSpec((_BI, _D), lambda i: (i, 0)),
        out_shape=jax.ShapeDtypeStruct((_N, _D), jnp.float32),
        scratch_shapes=[
            pltpu.VMEM((_N, _DH), jnp.float32),
            pltpu.VMEM((_N, _C), jnp.float32),
            pltpu.VMEM((_C, _N), jnp.float32),
            pltpu.VMEM((_C, _N), jnp.float32),
        ],
        compiler_params=pltpu.CompilerParams(
            dimension_semantics=("arbitrary",)),
    )(atom_embeddings, edges, Wq, bq_rep, Wk, bk_rep, Wv, bv2,
      asrc2, adst2, W_proj, bp2)
    return out


# speedup vs baseline: 1.8602x; 1.8602x over previous
import jax
import jax.numpy as jnp
from jax.experimental import pallas as pl
from jax.experimental.pallas import tpu as pltpu

_N = 2048
_D = 64
_H = 8
_NB = 8
_DH = _D * _H
_C = _H * _NB
_LOG2E = 1.4426950408889634
_SLOPE = 0.2
_BI = 512


def _gat_kernel(emb_ref, edges_ref, wq_ref, bqr_ref, wk_ref, bkr_ref,
                wv_ref, bv_ref, asrc_ref, adst_ref, wp_ref, bp_ref,
                out_ref, val_s, ssrc_s, sdstt_s, mb_s):
    step = pl.program_id(0)

    @pl.when(step == 0)
    def _init():
        emb = emb_ref[...]
        val_s[...] = (
            jnp.dot(emb, wv_ref[...], preferred_element_type=jnp.float32)
            + bv_ref[...])
        asrc = asrc_ref[...]
        adst = adst_ref[...]
        wq = wq_ref[...]
        wk = wk_ref[...]
        cq_blocks = []
        ck_blocks = []
        for h in range(_H):
            ah = asrc[h * _NB:(h + 1) * _NB, :]
            dh = adst[h * _NB:(h + 1) * _NB, :]
            wq_h = wq[:, h * _D:(h + 1) * _D]
            wk_h = wk[:, h * _D:(h + 1) * _D]
            cq_blocks.append(jax.lax.dot_general(
                wq_h, ah, (((1,), (1,)), ((), ())),
                preferred_element_type=jnp.float32))
            ck_blocks.append(jax.lax.dot_general(
                wk_h, dh, (((1,), (1,)), ((), ())),
                preferred_element_type=jnp.float32))
        cq = jnp.concatenate(cq_blocks, axis=1)
        ck = jnp.concatenate(ck_blocks, axis=1)
        cq_b = jnp.sum(bqr_ref[...] * asrc, axis=1, keepdims=True)
        ck_b = jnp.sum(bkr_ref[...] * adst, axis=1, keepdims=True)
        ssrc_s[...] = jnp.dot(emb, cq,
                              preferred_element_type=jnp.float32) * _LOG2E
        sdstt = (jax.lax.dot_general(ck, emb, (((0,), (1,)), ((), ())),
                                     preferred_element_type=jnp.float32)
                 + cq_b + ck_b) * _LOG2E
        sdstt_s[...] = sdstt
        mb_s[...] = jnp.concatenate(
            [jnp.max(sdstt[h * _NB:(h + 1) * _NB, :], keepdims=True)
             for h in range(_H)], axis=1)

    e = edges_ref[...]
    ssrc = ssrc_s[pl.ds(step * _BI, _BI), :]
    sdstt = sdstt_s[...]
    isrc = e & 15
    idst = e & 7
    neg_half = jnp.full((_BI, _NB), -7e8, jnp.float32)
    acc = jnp.zeros((_BI, _D), jnp.float32)
    for h in range(_H):
        ta = ssrc[:, h * _NB:(h + 1) * _NB]
        tb = sdstt[h * _NB:(h + 1) * _NB, :]
        bound = jnp.max(ta, axis=1, keepdims=True) + mb_s[0, h]
        bound = jnp.maximum(bound, _SLOPE * bound)
        ta16 = jnp.concatenate([ta, neg_half], axis=1)
        ga = jnp.take_along_axis(ta16, isrc, axis=1)
        gb = jnp.take_along_axis(tb, idst, axis=0)
        x = ga + gb
        x = jnp.maximum(x, _SLOPE * x)
        p = jnp.exp2(x - bound)
        inv = 1.0 / jnp.maximum(jnp.sum(p, axis=1, keepdims=True), 1e-30)
        oh = jnp.dot(p, val_s[:, h * _D:(h + 1) * _D],
                     preferred_element_type=jnp.float32) * inv
        acc = acc + jnp.dot(oh, wp_ref[h * _D:(h + 1) * _D, :],
                            preferred_element_type=jnp.float32)
    out_ref[...] = acc + bp_ref[...]


def kernel(atom_embeddings, edges, Wq, bq, Wk, bk, Wv, bv, a_src, a_dst,
           W_proj, b_proj):
    asrc2 = a_src.transpose(1, 0, 2).reshape(_C, _D)
    adst2 = a_dst.transpose(1, 0, 2).reshape(_C, _D)
    bq_rep = jnp.broadcast_to(
        bq.reshape(_H, 1, _D), (_H, _NB, _D)).reshape(_C, _D)
    bk_rep = jnp.broadcast_to(
        bk.reshape(_H, 1, _D), (_H, _NB, _D)).reshape(_C, _D)
    bv2 = bv.reshape(1, _DH)
    bp2 = b_proj.reshape(1, _D)

    full = lambda shape: pl.BlockSpec(shape, lambda i: (0,) * len(shape))
    out = pl.pallas_call(
        _gat_kernel,
        grid=(_N // _BI,),
        in_specs=[
            full((_N, _D)),
            pl.BlockSpec((_BI, _N), lambda i: (i, 0)),
            full((_D, _DH)),
            full((_C, _D)),
            full((_D, _DH)),
            full((_C, _D)),
            full((_D, _DH)),
            full((1, _DH)),
            full((_C, _D)),
            full((_C, _D)),
            full((_DH, _D)),
            full((1, _D)),
        ],
        out_specs=pl.BlockSpec((_BI, _D), lambda i: (i, 0)),
        out_shape=jax.ShapeDtypeStruct((_N, _D), jnp.float32),
        scratch_shapes=[
            pltpu.VMEM((_N, _DH), jnp.float32),
            pltpu.VMEM((_N, _C), jnp.float32),
            pltpu.VMEM((_C, _N), jnp.float32),
            pltpu.VMEM((1, _H), jnp.float32),
        ],
        compiler_params=pltpu.CompilerParams(
            dimension_semantics=("arbitrary",)),
    )(atom_embeddings, edges, Wq, bq_rep, Wk, bk_rep, Wv, bv2,
      asrc2, adst2, W_proj, bp2)
    return out
